# baseline (speedup 1.0000x reference)
import functools

import jax
import jax.numpy as jnp
from jax import lax
from jax.experimental import pallas as pl
from jax.experimental.pallas import tpu as pltpu
from jax.experimental.pallas import tpu_sc as plsc

B = 1024
S = 10
N1 = B * S
N2 = N1 * S
D = 128
DE = 16
NW = 32
PB = B // NW
P1 = N1 // NW
P2 = N2 // NW
CK = 80
N_NODES_ = 10000
F32 = jnp.float32
I32 = jnp.int32


NCH = P2 // CK
NP = NCH // 2
_SC_PARAMS = pltpu.CompilerParams(
    needs_layout_passes=False, use_tc_tiling_on_sc=False)


def _flatten10(dst, src, n, lanes):
    def it(i, _):
        g = i * 16 + lanes
        q = lax.shift_right_logical(g * 52429, 19)
        r = g - q * S
        dst[pl.ds(i * 16, 16)] = plsc.load_gather(src, [q, r])
        return 0
    lax.fori_loop(0, n // 16, it, 0)


def _seg_reduce(chunk_buf, stage, cbase, nvec):
    def it(d, _):
        for cv in range(nvec):
            acc = chunk_buf[d * S, pl.ds(cv * 16, 16)]
            for k in range(1, S):
                acc = acc + chunk_buf[d * S + k, pl.ds(cv * 16, 16)]
            stage[cbase + d, pl.ds(cv * 16, 16)] = acc
        return 0
    lax.fori_loop(0, CK // S, it, 0)


def _sc_feats_fn():
    mesh = plsc.VectorSubcoreMesh(core_axis_name="c", subcore_axis_name="s")
    out_type = [
        jax.ShapeDtypeStruct((B, D), F32),
        jax.ShapeDtypeStruct((N1, D), F32),
        jax.ShapeDtypeStruct((N1, D), F32),
        jax.ShapeDtypeStruct((N1, D), F32),
        jax.ShapeDtypeStruct((N1, D), F32),
        jax.ShapeDtypeStruct((N1,), I32),
        jax.ShapeDtypeStruct((N2,), I32),
        jax.ShapeDtypeStruct((N1,), I32),
        jax.ShapeDtypeStruct((N2,), I32),
    ]
    scratch = [
        pltpu.VMEM((PB,), I32),
        pltpu.VMEM((P1, 32), I32),
        pltpu.VMEM((P1, 32), I32),
        pltpu.VMEM((P1,), I32),
        pltpu.VMEM((P1,), I32),
        pltpu.VMEM((P2,), I32),
        pltpu.VMEM((P2,), I32),
        pltpu.VMEM((CK, D), F32),
        pltpu.VMEM((CK, D), F32),
        pltpu.VMEM((P1, D), F32),
    ] + [pltpu.SemaphoreType.DMA] * 3

    def body(ids_hbm, feats_hbm, adj0, adje0, adj1, adje1,
             g0_o, g1_0o, s2_0o, g1_1o, s2_1o, e1_0o, e2_0o, e1_1o, e2_1o,
             idsv, a2, ae2, cur1, e1v, n2, e2, FA, FB, fstage,
             sF, sE, sFB_):
        wid = lax.axis_index("s") * 2 + lax.axis_index("c")
        lanes = lax.iota(I32, 16)

        pltpu.sync_copy(ids_hbm.at[pl.ds(wid * PB, PB)], idsv)
        pltpu.async_copy(feats_hbm.at[idsv], FA.at[pl.ds(0, PB)], sF).wait()
        pltpu.sync_copy(FA.at[pl.ds(0, PB)], g0_o.at[pl.ds(wid * PB, PB)])

        for adj, adje, g1_o, s2_o, e1_o, e2_o in (
            (adj0, adje0, g1_0o, s2_0o, e1_0o, e2_0o),
            (adj1, adje1, g1_1o, s2_1o, e1_1o, e2_1o),
        ):
            h1a = pltpu.async_copy(adj.at[idsv], a2.at[pl.ds(0, PB)], sF)
            h1b = pltpu.async_copy(adje.at[idsv], ae2.at[pl.ds(0, PB)], sE)
            h1a.wait()
            _flatten10(cur1, a2, P1, lanes)
            h1b.wait()
            _flatten10(e1v, ae2, P1, lanes)
            pltpu.sync_copy(e1v, e1_o.at[pl.ds(wid * P1, P1)])

            ha, he, hf = [], [], []
            for c in range(P1 // CK):
                sl = pl.ds(c * CK, CK)
                ha.append(pltpu.async_copy(adj.at[cur1.at[sl]], a2.at[sl], sF))
                he.append(pltpu.async_copy(adje.at[cur1.at[sl]], ae2.at[sl], sE))
                hf.append(pltpu.async_copy(feats_hbm.at[cur1.at[sl]],
                                           fstage.at[sl], sFB_))
            for h in ha:
                h.wait()
            _flatten10(n2, a2, P2, lanes)
            for h in he:
                h.wait()
            _flatten10(e2, ae2, P2, lanes)
            pltpu.sync_copy(e2, e2_o.at[pl.ds(wid * P2, P2)])
            for h in hf:
                h.wait()
            pltpu.sync_copy(fstage, g1_o.at[pl.ds(wid * P1, P1)])

            def g_f(c, buf, sem):
                return pltpu.async_copy(
                    feats_hbm.at[n2.at[pl.ds(c * CK, CK)]], buf, sem)

            g_f(0, FA, sF)

            def pair(i, _):
                c0 = 2 * i
                c1 = 2 * i + 1
                hfb = g_f(c1, FB, sFB_)
                pltpu.make_async_copy(
                    feats_hbm.at[n2.at[pl.ds(0, CK)]], FA, sF).wait()
                _seg_reduce(FA, fstage, c0 * (CK // S), D // 16)

                @pl.when(i < NP - 1)
                def _():
                    g_f(c0 + 2, FA, sF)

                hfb.wait()
                _seg_reduce(FB, fstage, c1 * (CK // S), D // 16)
                return 0

            lax.fori_loop(0, NP, pair, 0)
            pltpu.sync_copy(fstage, s2_o.at[pl.ds(wid * P1, P1)])

    return functools.partial(
        pl.kernel, mesh=mesh, out_type=out_type, scratch_types=scratch,
        compiler_params=_SC_PARAMS)(body)


def _sc_edges_fn():
    mesh = plsc.VectorSubcoreMesh(core_axis_name="c", subcore_axis_name="s")
    out_type = [
        jax.ShapeDtypeStruct((N1, DE), F32),
        jax.ShapeDtypeStruct((N1, DE), F32),
    ]
    scratch = [
        pltpu.VMEM((P1,), I32),
        pltpu.VMEM((P2,), I32),
        pltpu.VMEM((CK, DE), F32),
        pltpu.VMEM((CK, DE), F32),
        pltpu.VMEM((P1, DE), F32),
    ] + [pltpu.SemaphoreType.DMA] * 3

    def body(emb, e1_i, e2_i, e1_o, se2_o, e1v, e2, EA, EB, estage,
             sF, sE, sFB_):
        wid = lax.axis_index("s") * 2 + lax.axis_index("c")

        pltpu.sync_copy(e1_i.at[pl.ds(wid * P1, P1)], e1v)
        h2 = pltpu.async_copy(e2_i.at[pl.ds(wid * P2, P2)], e2, sE)
        hg = []
        for c in range(P1 // CK):
            sl = pl.ds(c * CK, CK)
            hg.append(pltpu.async_copy(emb.at[e1v.at[sl]],
                                       estage.at[sl], sFB_))
        h2.wait()
        for h in hg:
            h.wait()
        pltpu.sync_copy(estage, e1_o.at[pl.ds(wid * P1, P1)])

        def g_e(c, buf, sem):
            return pltpu.async_copy(
                emb.at[e2.at[pl.ds(c * CK, CK)]], buf, sem)

        g_e(0, EA, sF)

        def pair(i, _):
            c0 = 2 * i
            c1 = 2 * i + 1
            heb = g_e(c1, EB, sFB_)
            pltpu.make_async_copy(
                emb.at[e2.at[pl.ds(0, CK)]], EA, sF).wait()
            _seg_reduce(EA, estage, c0 * (CK // S), DE // 16)

            @pl.when(i < NP - 1)
            def _():
                g_e(c0 + 2, EA, sF)

            heb.wait()
            _seg_reduce(EB, estage, c1 * (CK // S), DE // 16)
            return 0

        lax.fori_loop(0, NP, pair, 0)
        pltpu.sync_copy(estage, se2_o.at[pl.ds(wid * P1, P1)])

    return functools.partial(
        pl.kernel, mesh=mesh, out_type=out_type, scratch_types=scratch,
        compiler_params=_SC_PARAMS)(body)


def _repack_body(x_ref, o_ref):
    Rw, CB = x_ref.shape
    x = x_ref[...]
    if x.dtype != F32:
        x = x.astype(F32)
    ident = jnp.eye(Rw, dtype=F32)
    t = lax.dot_general(x, ident, (((0,), (0,)), ((), ())),
                        precision=lax.Precision.HIGHEST,
                        preferred_element_type=F32)
    if o_ref.dtype != F32:
        t = t.astype(o_ref.dtype)
    t3 = t.reshape(CB // 8, 8, Rw)
    o_ref[...] = jnp.concatenate([t3[:, s, :] for s in range(8)], axis=1)


def _prep_transpose(xT, blk_c):
    Rw, C = xT.shape
    nsteps = C // blk_c
    return pl.pallas_call(
        _repack_body,
        grid=(nsteps,),
        in_specs=[pl.BlockSpec((Rw, blk_c), lambda i: (0, i))],
        out_specs=pl.BlockSpec((blk_c // 8, Rw * 8), lambda i: (i, 0)),
        out_shape=jax.ShapeDtypeStruct((C // 8, Rw * 8), xT.dtype),
    )(xT)


def _prep_transpose4(xTs):
    Rw, C = xTs[0].shape

    def body(x0, x1, x2, x3, o0, o1, o2, o3):
        for x_ref, o_ref in ((x0, o0), (x1, o1), (x2, o2), (x3, o3)):
            _repack_body(x_ref, o_ref)

    outs = pl.pallas_call(
        body,
        out_shape=[jax.ShapeDtypeStruct((C // 8, Rw * 8), x.dtype)
                   for x in xTs],
    )(*xTs)
    return outs


BB = 128
GRID = B // BB


def _tc_body(g0, g1_0, s2_0, e1_0, se2_0, g1_1, s2_1, e1_1, se2_1,
             Wp, bp,
             Ws00, Wn00, ba00, We00, be00, Ws01, Wn01, ba01,
             Ws10, Wn10, ba10, We10, be10, Ws11, Wn11, ba11,
             Wfc, bfc, out):
    inv_s = 1.0 / S

    def gmean(x):
        return jnp.sum(x.reshape(BB, S, x.shape[-1]), axis=1) * inv_s

    def grepeat(x):
        return jnp.broadcast_to(x[:, None, :], (BB, S, x.shape[-1])).reshape(BB * S, x.shape[-1])

    wp = Wp[...]
    bpv = bp[...]
    p0 = jnp.dot(g0[...], wp, preferred_element_type=F32) + bpv

    acc = None
    for g1, s2, e1, se2, Ws0, Wn0, ba0, We0, be0, Ws1, Wn1, ba1 in (
        (g1_0, s2_0, e1_0, se2_0, Ws00, Wn00, ba00, We00, be00, Ws01, Wn01, ba01),
        (g1_1, s2_1, e1_1, se2_1, Ws10, Wn10, ba10, We10, be10, Ws11, Wn11, ba11),
    ):
        wn0 = Wn0[...]
        wn1 = Wn1[...]
        we0 = We0[...]
        p1 = jnp.dot(g1[...], wp, preferred_element_type=F32) + bpv
        m2 = jnp.dot(s2[...] * inv_s, wp, preferred_element_type=F32) + bpv
        se2m = se2[...] * inv_s
        e1v = e1[...]
        f1 = jnp.maximum(
            jnp.dot(p1, Ws0[...], preferred_element_type=F32)
            + jnp.dot(m2, wn0[:D], preferred_element_type=F32)
            + jnp.dot(se2m, wn0[D:], preferred_element_type=F32) + ba0[...], 0.0)
        f0 = jnp.maximum(
            jnp.dot(p0, Ws0[...], preferred_element_type=F32)
            + jnp.dot(gmean(p1), wn0[:D], preferred_element_type=F32)
            + jnp.dot(gmean(e1v), wn0[D:], preferred_element_type=F32) + ba0[...], 0.0)
        e0n = jnp.maximum(
            jnp.dot(grepeat(f0), we0[:D], preferred_element_type=F32)
            + jnp.dot(f1, we0[D:2 * D], preferred_element_type=F32)
            + jnp.dot(e1v, we0[2 * D:], preferred_element_type=F32) + be0[...], 0.0)
        o = jnp.maximum(
            jnp.dot(f0, Ws1[...], preferred_element_type=F32)
            + jnp.dot(gmean(f1), wn1[:D], preferred_element_type=F32)
            + jnp.dot(gmean(e0n), wn1[D:], preferred_element_type=F32) + ba1[...], 0.0)
        acc = o if acc is None else acc + o

    res = acc * 0.5
    nrm = jnp.sqrt(jnp.sum(res * res, axis=1, keepdims=True))
    res = res / jnp.maximum(nrm, 1e-12)
    out[...] = jnp.dot(res, Wfc[...], preferred_element_type=F32) + bfc[...]


def _tc_call(g0, g1_0, s2_0, e1_0, se2_0, g1_1, s2_1, e1_1, se2_1, *ws):
    bspec_b = pl.BlockSpec((BB, D), lambda i: (i, 0))
    bspec_n = pl.BlockSpec((BB * S, D), lambda i: (i, 0))
    bspec_be = pl.BlockSpec((BB * S, DE), lambda i: (i, 0))
    full = lambda a: pl.BlockSpec(a.shape, lambda i: tuple(0 for _ in a.shape))
    in_specs = [bspec_b, bspec_n, bspec_n, bspec_be, bspec_be,
                bspec_n, bspec_n, bspec_be, bspec_be] + [full(w) for w in ws]
    return pl.pallas_call(
        _tc_body,
        grid=(GRID,),
        in_specs=in_specs,
        out_specs=pl.BlockSpec((BB, 8), lambda i: (i, 0)),
        out_shape=jax.ShapeDtypeStruct((B, 8), F32),
    )(g0, g1_0, s2_0, e1_0, se2_0, g1_1, s2_1, e1_1, se2_1, *ws)


def kernel(ids, feats, adj_0, adj_edge_0, edge_emb_0, adj_1, adj_edge_1,
           edge_emb_1, W_prep, b_prep,
           W_self_0_0, W_neigh_0_0, b_agg_0_0, W_edge_0_0, b_edge_0_0,
           W_self_0_1, W_neigh_0_1, b_agg_0_1, W_edge_0_1, b_edge_0_1,
           W_self_1_0, W_neigh_1_0, b_agg_1_0, W_edge_1_0, b_edge_1_0,
           W_self_1_1, W_neigh_1_1, b_agg_1_1, W_edge_1_1, b_edge_1_1,
           W_fc, b_fc):
    def repack(t, blk_c):
        n, w = t.shape
        return _prep_transpose(jnp.swapaxes(t, 0, 1), blk_c).reshape(n, w)

    adjs = _prep_transpose4([jnp.swapaxes(t, 0, 1) for t in
                             (adj_0, adj_edge_0, adj_1, adj_edge_1)])
    adj_0c, adje_0c, adj_1c, adje_1c = (
        t.reshape(N_NODES_, 32) for t in adjs)
    emb_0c = repack(edge_emb_0, 6400)
    emb_1c = repack(edge_emb_1, 6400)

    sc_a = _sc_feats_fn()
    (g0, g1_0, s2_0, g1_1, s2_1, e1i_0, e2i_0, e1i_1, e2i_1) = sc_a(
        ids, feats, adj_0c, adje_0c, adj_1c, adje_1c)
    sc_b = _sc_edges_fn()
    (e1_0, se2_0) = sc_b(emb_0c, e1i_0, e2i_0)
    (e1_1, se2_1) = sc_b(emb_1c, e1i_1, e2i_1)
    return _tc_call(
        g0, g1_0, s2_0, e1_0, se2_0, g1_1, s2_1, e1_1, se2_1,
        W_prep, b_prep,
        W_self_0_0, W_neigh_0_0, b_agg_0_0, W_edge_0_0, b_edge_0_0,
        W_self_0_1, W_neigh_0_1, b_agg_0_1,
        W_self_1_0, W_neigh_1_0, b_agg_1_0, W_edge_1_0, b_edge_1_0,
        W_self_1_1, W_neigh_1_1, b_agg_1_1,
        W_fc, b_fc)


# speedup vs baseline: 1.6933x; 1.6933x over previous
import functools

import jax
import jax.numpy as jnp
from jax import lax
from jax.experimental import pallas as pl
from jax.experimental.pallas import tpu as pltpu
from jax.experimental.pallas import tpu_sc as plsc

B = 1024
S = 10
N1 = B * S
N2 = N1 * S
D = 128
DE = 16
NW = 32
PB = B // NW
P1 = N1 // NW
P2 = N2 // NW
CK = 80
N_NODES_ = 10000
F32 = jnp.float32
I32 = jnp.int32


NCH = P2 // CK
NP = NCH // 2
_SC_PARAMS = pltpu.CompilerParams(
    needs_layout_passes=False, use_tc_tiling_on_sc=False)


def _flatten10(dst, src, n, lanes):
    def it(i, _):
        g = i * 16 + lanes
        q = lax.shift_right_logical(g * 52429, 19)
        r = g - q * S
        dst[pl.ds(i * 16, 16)] = plsc.load_gather(src, [q, r])
        return 0
    lax.fori_loop(0, n // 16, it, 0)


def _seg_reduce(chunk_buf, stage, cbase, nvec):
    def it(d, _):
        for cv in range(nvec):
            acc = chunk_buf[d * S, pl.ds(cv * 16, 16)]
            for k in range(1, S):
                acc = acc + chunk_buf[d * S + k, pl.ds(cv * 16, 16)]
            stage[cbase + d, pl.ds(cv * 16, 16)] = acc
        return 0
    lax.fori_loop(0, CK // S, it, 0)


def _sc_feats_fn():
    mesh = plsc.VectorSubcoreMesh(core_axis_name="c", subcore_axis_name="s")
    out_type = [
        jax.ShapeDtypeStruct((B, D), F32),
        jax.ShapeDtypeStruct((N1, D), F32),
        jax.ShapeDtypeStruct((N1, D), F32),
        jax.ShapeDtypeStruct((N1, D), F32),
        jax.ShapeDtypeStruct((N1, D), F32),
        jax.ShapeDtypeStruct((N1,), I32),
        jax.ShapeDtypeStruct((N2,), I32),
        jax.ShapeDtypeStruct((N1,), I32),
        jax.ShapeDtypeStruct((N2,), I32),
    ]
    scratch = [
        pltpu.VMEM((PB,), I32),
        pltpu.VMEM((P1, 32), I32),
        pltpu.VMEM((P1, 32), I32),
        pltpu.VMEM((P1,), I32),
        pltpu.VMEM((P1,), I32),
        pltpu.VMEM((P2,), I32),
        pltpu.VMEM((P2,), I32),
        pltpu.VMEM((CK, D), F32),
        pltpu.VMEM((CK, D), F32),
        pltpu.VMEM((P1, D), F32),
    ] + [pltpu.SemaphoreType.DMA] * 3

    def body(ids_hbm, feats_hbm, adj0, adje0, adj1, adje1,
             g0_o, g1_0o, s2_0o, g1_1o, s2_1o, e1_0o, e2_0o, e1_1o, e2_1o,
             idsv, a2, ae2, cur1, e1v, n2, e2, FA, FB, fstage,
             sF, sE, sFB_):
        wid = lax.axis_index("s") * 2 + lax.axis_index("c")
        lanes = lax.iota(I32, 16)

        pltpu.sync_copy(ids_hbm.at[pl.ds(wid * PB, PB)], idsv)
        pltpu.async_copy(feats_hbm.at[idsv], FA.at[pl.ds(0, PB)], sF).wait()
        pltpu.sync_copy(FA.at[pl.ds(0, PB)], g0_o.at[pl.ds(wid * PB, PB)])

        for adj, adje, g1_o, s2_o, e1_o, e2_o in (
            (adj0, adje0, g1_0o, s2_0o, e1_0o, e2_0o),
            (adj1, adje1, g1_1o, s2_1o, e1_1o, e2_1o),
        ):
            h1a = pltpu.async_copy(adj.at[idsv], a2.at[pl.ds(0, PB)], sF)
            h1b = pltpu.async_copy(adje.at[idsv], ae2.at[pl.ds(0, PB)], sE)
            h1a.wait()
            _flatten10(cur1, a2, P1, lanes)
            h1b.wait()
            _flatten10(e1v, ae2, P1, lanes)
            pltpu.sync_copy(e1v, e1_o.at[pl.ds(wid * P1, P1)])

            ha, he, hf = [], [], []
            for c in range(P1 // CK):
                sl = pl.ds(c * CK, CK)
                ha.append(pltpu.async_copy(adj.at[cur1.at[sl]], a2.at[sl], sF))
                he.append(pltpu.async_copy(adje.at[cur1.at[sl]], ae2.at[sl], sE))
                hf.append(pltpu.async_copy(feats_hbm.at[cur1.at[sl]],
                                           fstage.at[sl], sFB_))
            for h in ha:
                h.wait()
            _flatten10(n2, a2, P2, lanes)
            for h in he:
                h.wait()
            _flatten10(e2, ae2, P2, lanes)
            pltpu.sync_copy(e2, e2_o.at[pl.ds(wid * P2, P2)])
            for h in hf:
                h.wait()
            pltpu.sync_copy(fstage, g1_o.at[pl.ds(wid * P1, P1)])

            def g_f(c, buf, sem):
                return pltpu.async_copy(
                    feats_hbm.at[n2.at[pl.ds(c * CK, CK)]], buf, sem)

            g_f(0, FA, sF)

            def pair(i, _):
                c0 = 2 * i
                c1 = 2 * i + 1
                hfb = g_f(c1, FB, sFB_)
                pltpu.make_async_copy(
                    feats_hbm.at[n2.at[pl.ds(0, CK)]], FA, sF).wait()
                _seg_reduce(FA, fstage, c0 * (CK // S), D // 16)

                @pl.when(i < NP - 1)
                def _():
                    g_f(c0 + 2, FA, sF)

                hfb.wait()
                _seg_reduce(FB, fstage, c1 * (CK // S), D // 16)
                return 0

            lax.fori_loop(0, NP, pair, 0)
            pltpu.sync_copy(fstage, s2_o.at[pl.ds(wid * P1, P1)])

    return functools.partial(
        pl.kernel, mesh=mesh, out_type=out_type, scratch_types=scratch,
        compiler_params=_SC_PARAMS)(body)


def _sc_edges_fn():
    mesh = plsc.VectorSubcoreMesh(core_axis_name="c", subcore_axis_name="s")
    out_type = [
        jax.ShapeDtypeStruct((N1, DE), F32),
        jax.ShapeDtypeStruct((N1, DE), F32),
    ]
    scratch = [
        pltpu.VMEM((P1,), I32),
        pltpu.VMEM((P2,), I32),
        pltpu.VMEM((CK, DE), F32),
        pltpu.VMEM((CK, DE), F32),
        pltpu.VMEM((P1, DE), F32),
    ] + [pltpu.SemaphoreType.DMA] * 3

    def body(emb, e1_i, e2_i, e1_o, se2_o, e1v, e2, EA, EB, estage,
             sF, sE, sFB_):
        wid = lax.axis_index("s") * 2 + lax.axis_index("c")

        pltpu.sync_copy(e1_i.at[pl.ds(wid * P1, P1)], e1v)
        h2 = pltpu.async_copy(e2_i.at[pl.ds(wid * P2, P2)], e2, sE)
        hg = []
        for c in range(P1 // CK):
            sl = pl.ds(c * CK, CK)
            hg.append(pltpu.async_copy(emb.at[e1v.at[sl]],
                                       estage.at[sl], sFB_))
        h2.wait()
        for h in hg:
            h.wait()
        pltpu.sync_copy(estage, e1_o.at[pl.ds(wid * P1, P1)])

        def g_e(c, buf, sem):
            return pltpu.async_copy(
                emb.at[e2.at[pl.ds(c * CK, CK)]], buf, sem)

        g_e(0, EA, sF)

        def pair(i, _):
            c0 = 2 * i
            c1 = 2 * i + 1
            heb = g_e(c1, EB, sFB_)
            pltpu.make_async_copy(
                emb.at[e2.at[pl.ds(0, CK)]], EA, sF).wait()
            _seg_reduce(EA, estage, c0 * (CK // S), DE // 16)

            @pl.when(i < NP - 1)
            def _():
                g_e(c0 + 2, EA, sF)

            heb.wait()
            _seg_reduce(EB, estage, c1 * (CK // S), DE // 16)
            return 0

        lax.fori_loop(0, NP, pair, 0)
        pltpu.sync_copy(estage, se2_o.at[pl.ds(wid * P1, P1)])

    return functools.partial(
        pl.kernel, mesh=mesh, out_type=out_type, scratch_types=scratch,
        compiler_params=_SC_PARAMS)(body)


def _repack_body(x_ref, o_ref):
    Rw, CB = x_ref.shape
    t3 = x_ref[...].T.reshape(CB // 8, 8, Rw)
    for s in range(8):
        o_ref[:, s * Rw:(s + 1) * Rw] = t3[:, s, :]


def _prep_transpose(xT, blk_c):
    Rw, C = xT.shape
    nsteps = C // blk_c
    return pl.pallas_call(
        _repack_body,
        grid=(nsteps,),
        in_specs=[pl.BlockSpec((Rw, blk_c), lambda i: (0, i))],
        out_specs=pl.BlockSpec((blk_c // 8, Rw * 8), lambda i: (i, 0)),
        out_shape=jax.ShapeDtypeStruct((C // 8, Rw * 8), xT.dtype),
    )(xT)


def _prep_transpose4(xTs):
    Rw, C = xTs[0].shape

    def body(x0, x1, x2, x3, o0, o1, o2, o3):
        for x_ref, o_ref in ((x0, o0), (x1, o1), (x2, o2), (x3, o3)):
            _repack_body(x_ref, o_ref)

    outs = pl.pallas_call(
        body,
        out_shape=[jax.ShapeDtypeStruct((C // 8, Rw * 8), x.dtype)
                   for x in xTs],
    )(*xTs)
    return outs


BB = 128
GRID = B // BB


def _tc_body(g0, g1_0, s2_0, e1_0, se2_0, g1_1, s2_1, e1_1, se2_1,
             Wp, bp,
             Ws00, Wn00, ba00, We00, be00, Ws01, Wn01, ba01,
             Ws10, Wn10, ba10, We10, be10, Ws11, Wn11, ba11,
             Wfc, bfc, out):
    inv_s = 1.0 / S

    def gmean(x):
        return jnp.sum(x.reshape(BB, S, x.shape[-1]), axis=1) * inv_s

    def grepeat(x):
        return jnp.broadcast_to(x[:, None, :], (BB, S, x.shape[-1])).reshape(BB * S, x.shape[-1])

    wp = Wp[...]
    bpv = bp[...]
    p0 = jnp.dot(g0[...], wp, preferred_element_type=F32) + bpv

    acc = None
    for g1, s2, e1, se2, Ws0, Wn0, ba0, We0, be0, Ws1, Wn1, ba1 in (
        (g1_0, s2_0, e1_0, se2_0, Ws00, Wn00, ba00, We00, be00, Ws01, Wn01, ba01),
        (g1_1, s2_1, e1_1, se2_1, Ws10, Wn10, ba10, We10, be10, Ws11, Wn11, ba11),
    ):
        wn0 = Wn0[...]
        wn1 = Wn1[...]
        we0 = We0[...]
        p1 = jnp.dot(g1[...], wp, preferred_element_type=F32) + bpv
        m2 = jnp.dot(s2[...] * inv_s, wp, preferred_element_type=F32) + bpv
        se2m = se2[...] * inv_s
        e1v = e1[...]
        f1 = jnp.maximum(
            jnp.dot(p1, Ws0[...], preferred_element_type=F32)
            + jnp.dot(m2, wn0[:D], preferred_element_type=F32)
            + jnp.dot(se2m, wn0[D:], preferred_element_type=F32) + ba0[...], 0.0)
        f0 = jnp.maximum(
            jnp.dot(p0, Ws0[...], preferred_element_type=F32)
            + jnp.dot(gmean(p1), wn0[:D], preferred_element_type=F32)
            + jnp.dot(gmean(e1v), wn0[D:], preferred_element_type=F32) + ba0[...], 0.0)
        e0n = jnp.maximum(
            jnp.dot(grepeat(f0), we0[:D], preferred_element_type=F32)
            + jnp.dot(f1, we0[D:2 * D], preferred_element_type=F32)
            + jnp.dot(e1v, we0[2 * D:], preferred_element_type=F32) + be0[...], 0.0)
        o = jnp.maximum(
            jnp.dot(f0, Ws1[...], preferred_element_type=F32)
            + jnp.dot(gmean(f1), wn1[:D], preferred_element_type=F32)
            + jnp.dot(gmean(e0n), wn1[D:], preferred_element_type=F32) + ba1[...], 0.0)
        acc = o if acc is None else acc + o

    res = acc * 0.5
    nrm = jnp.sqrt(jnp.sum(res * res, axis=1, keepdims=True))
    res = res / jnp.maximum(nrm, 1e-12)
    out[...] = jnp.dot(res, Wfc[...], preferred_element_type=F32) + bfc[...]


def _tc_call(g0, g1_0, s2_0, e1_0, se2_0, g1_1, s2_1, e1_1, se2_1, *ws):
    bspec_b = pl.BlockSpec((BB, D), lambda i: (i, 0))
    bspec_n = pl.BlockSpec((BB * S, D), lambda i: (i, 0))
    bspec_be = pl.BlockSpec((BB * S, DE), lambda i: (i, 0))
    full = lambda a: pl.BlockSpec(a.shape, lambda i: tuple(0 for _ in a.shape))
    in_specs = [bspec_b, bspec_n, bspec_n, bspec_be, bspec_be,
                bspec_n, bspec_n, bspec_be, bspec_be] + [full(w) for w in ws]
    return pl.pallas_call(
        _tc_body,
        grid=(GRID,),
        in_specs=in_specs,
        out_specs=pl.BlockSpec((BB, 8), lambda i: (i, 0)),
        out_shape=jax.ShapeDtypeStruct((B, 8), F32),
    )(g0, g1_0, s2_0, e1_0, se2_0, g1_1, s2_1, e1_1, se2_1, *ws)


def kernel(ids, feats, adj_0, adj_edge_0, edge_emb_0, adj_1, adj_edge_1,
           edge_emb_1, W_prep, b_prep,
           W_self_0_0, W_neigh_0_0, b_agg_0_0, W_edge_0_0, b_edge_0_0,
           W_self_0_1, W_neigh_0_1, b_agg_0_1, W_edge_0_1, b_edge_0_1,
           W_self_1_0, W_neigh_1_0, b_agg_1_0, W_edge_1_0, b_edge_1_0,
           W_self_1_1, W_neigh_1_1, b_agg_1_1, W_edge_1_1, b_edge_1_1,
           W_fc, b_fc):
    def repack(t, blk_c):
        n, w = t.shape
        return _prep_transpose(jnp.swapaxes(t, 0, 1), blk_c).reshape(n, w)

    adjs = _prep_transpose4([jnp.swapaxes(t, 0, 1) for t in
                             (adj_0, adj_edge_0, adj_1, adj_edge_1)])
    adj_0c, adje_0c, adj_1c, adje_1c = (
        t.reshape(N_NODES_, 32) for t in adjs)
    emb_0c = repack(edge_emb_0, 6400)
    emb_1c = repack(edge_emb_1, 6400)

    sc_a = _sc_feats_fn()
    (g0, g1_0, s2_0, g1_1, s2_1, e1i_0, e2i_0, e1i_1, e2i_1) = sc_a(
        ids, feats, adj_0c, adje_0c, adj_1c, adje_1c)
    sc_b = _sc_edges_fn()
    (e1_0, se2_0) = sc_b(emb_0c, e1i_0, e2i_0)
    (e1_1, se2_1) = sc_b(emb_1c, e1i_1, e2i_1)
    return _tc_call(
        g0, g1_0, s2_0, e1_0, se2_0, g1_1, s2_1, e1_1, se2_1,
        W_prep, b_prep,
        W_self_0_0, W_neigh_0_0, b_agg_0_0, W_edge_0_0, b_edge_0_0,
        W_self_0_1, W_neigh_0_1, b_agg_0_1,
        W_self_1_0, W_neigh_1_0, b_agg_1_0, W_edge_1_0, b_edge_1_0,
        W_self_1_1, W_neigh_1_1, b_agg_1_1,
        W_fc, b_fc)


# speedup vs baseline: 1.6955x; 1.0013x over previous
import functools

import jax
import jax.numpy as jnp
from jax import lax
from jax.experimental import pallas as pl
from jax.experimental.pallas import tpu as pltpu
from jax.experimental.pallas import tpu_sc as plsc

B = 1024
S = 10
N1 = B * S
N2 = N1 * S
D = 128
DE = 16
NW = 32
PB = B // NW
P1 = N1 // NW
P2 = N2 // NW
CK = 80
N_NODES_ = 10000
F32 = jnp.float32
I32 = jnp.int32


NCH = P2 // CK
NP = NCH // 2
_SC_PARAMS = pltpu.CompilerParams(
    needs_layout_passes=False, use_tc_tiling_on_sc=False)


def _flatten10(dst, src, n, lanes):
    def it(i, _):
        g = i * 16 + lanes
        q = lax.shift_right_logical(g * 52429, 19)
        r = g - q * S
        dst[pl.ds(i * 16, 16)] = plsc.load_gather(src, [q, r])
        return 0
    lax.fori_loop(0, n // 16, it, 0)


def _seg_reduce(chunk_buf, stage, cbase, nvec):
    def it(d, _):
        for cv in range(nvec):
            acc = chunk_buf[d * S, pl.ds(cv * 16, 16)]
            for k in range(1, S):
                acc = acc + chunk_buf[d * S + k, pl.ds(cv * 16, 16)]
            stage[cbase + d, pl.ds(cv * 16, 16)] = acc
        return 0
    lax.fori_loop(0, CK // S, it, 0)


def _sc_feats_fn():
    mesh = plsc.VectorSubcoreMesh(core_axis_name="c", subcore_axis_name="s")
    out_type = [
        jax.ShapeDtypeStruct((B, D), F32),
        jax.ShapeDtypeStruct((N1, D), F32),
        jax.ShapeDtypeStruct((N1, D), F32),
        jax.ShapeDtypeStruct((N1, D), F32),
        jax.ShapeDtypeStruct((N1, D), F32),
        jax.ShapeDtypeStruct((N1,), I32),
        jax.ShapeDtypeStruct((N2,), I32),
        jax.ShapeDtypeStruct((N1,), I32),
        jax.ShapeDtypeStruct((N2,), I32),
    ]
    scratch = [
        pltpu.VMEM((PB,), I32),
        pltpu.VMEM((P1, 32), I32),
        pltpu.VMEM((P1, 32), I32),
        pltpu.VMEM((P1,), I32),
        pltpu.VMEM((P1,), I32),
        pltpu.VMEM((P2,), I32),
        pltpu.VMEM((P2,), I32),
        pltpu.VMEM((CK, D), F32),
        pltpu.VMEM((CK, D), F32),
        pltpu.VMEM((P1, D), F32),
    ] + [pltpu.SemaphoreType.DMA] * 3

    def body(ids_hbm, feats_hbm, adj0, adje0, adj1, adje1,
             g0_o, g1_0o, s2_0o, g1_1o, s2_1o, e1_0o, e2_0o, e1_1o, e2_1o,
             idsv, a2, ae2, cur1, e1v, n2, e2, FA, FB, fstage,
             sF, sE, sFB_):
        wid = lax.axis_index("s") * 2 + lax.axis_index("c")
        lanes = lax.iota(I32, 16)

        pltpu.sync_copy(ids_hbm.at[pl.ds(wid * PB, PB)], idsv)
        pltpu.async_copy(feats_hbm.at[idsv], FA.at[pl.ds(0, PB)], sF).wait()
        pltpu.sync_copy(FA.at[pl.ds(0, PB)], g0_o.at[pl.ds(wid * PB, PB)])

        for adj, adje, g1_o, s2_o, e1_o, e2_o in (
            (adj0, adje0, g1_0o, s2_0o, e1_0o, e2_0o),
            (adj1, adje1, g1_1o, s2_1o, e1_1o, e2_1o),
        ):
            h1a = pltpu.async_copy(adj.at[idsv], a2.at[pl.ds(0, PB)], sF)
            h1b = pltpu.async_copy(adje.at[idsv], ae2.at[pl.ds(0, PB)], sE)
            h1a.wait()
            _flatten10(cur1, a2, P1, lanes)
            h1b.wait()
            _flatten10(e1v, ae2, P1, lanes)
            pltpu.sync_copy(e1v, e1_o.at[pl.ds(wid * P1, P1)])

            ha, he, hf = [], [], []
            for c in range(P1 // CK):
                sl = pl.ds(c * CK, CK)
                ha.append(pltpu.async_copy(adj.at[cur1.at[sl]], a2.at[sl], sF))
                he.append(pltpu.async_copy(adje.at[cur1.at[sl]], ae2.at[sl], sE))
                hf.append(pltpu.async_copy(feats_hbm.at[cur1.at[sl]],
                                           fstage.at[sl], sFB_))
            for h in ha:
                h.wait()
            _flatten10(n2, a2, P2, lanes)
            for h in he:
                h.wait()
            _flatten10(e2, ae2, P2, lanes)
            pltpu.sync_copy(e2, e2_o.at[pl.ds(wid * P2, P2)])
            for h in hf:
                h.wait()
            pltpu.sync_copy(fstage, g1_o.at[pl.ds(wid * P1, P1)])

            def g_f(c, buf, sem):
                return pltpu.async_copy(
                    feats_hbm.at[n2.at[pl.ds(c * CK, CK)]], buf, sem)

            g_f(0, FA, sF)

            def pair(i, _):
                c0 = 2 * i
                c1 = 2 * i + 1
                hfb = g_f(c1, FB, sFB_)
                pltpu.make_async_copy(
                    feats_hbm.at[n2.at[pl.ds(0, CK)]], FA, sF).wait()
                _seg_reduce(FA, fstage, c0 * (CK // S), D // 16)

                @pl.when(i < NP - 1)
                def _():
                    g_f(c0 + 2, FA, sF)

                hfb.wait()
                _seg_reduce(FB, fstage, c1 * (CK // S), D // 16)
                return 0

            lax.fori_loop(0, NP, pair, 0)
            pltpu.sync_copy(fstage, s2_o.at[pl.ds(wid * P1, P1)])

    return functools.partial(
        pl.kernel, mesh=mesh, out_type=out_type, scratch_types=scratch,
        compiler_params=_SC_PARAMS)(body)


def _sc_edges_fn():
    mesh = plsc.VectorSubcoreMesh(core_axis_name="c", subcore_axis_name="s")
    out_type = [
        jax.ShapeDtypeStruct((N1, DE), F32),
        jax.ShapeDtypeStruct((N1, DE), F32),
    ]
    scratch = [
        pltpu.VMEM((P1,), I32),
        pltpu.VMEM((P2,), I32),
        pltpu.VMEM((CK, DE), F32),
        pltpu.VMEM((CK, DE), F32),
        pltpu.VMEM((P1, DE), F32),
    ] + [pltpu.SemaphoreType.DMA] * 3

    def body(emb, e1_i, e2_i, e1_o, se2_o, e1v, e2, EA, EB, estage,
             sF, sE, sFB_):
        wid = lax.axis_index("s") * 2 + lax.axis_index("c")

        pltpu.sync_copy(e1_i.at[pl.ds(wid * P1, P1)], e1v)
        h2 = pltpu.async_copy(e2_i.at[pl.ds(wid * P2, P2)], e2, sE)
        hg = []
        for c in range(P1 // CK):
            sl = pl.ds(c * CK, CK)
            hg.append(pltpu.async_copy(emb.at[e1v.at[sl]],
                                       estage.at[sl], sFB_))
        h2.wait()
        for h in hg:
            h.wait()
        pltpu.sync_copy(estage, e1_o.at[pl.ds(wid * P1, P1)])

        def g_e(c, buf, sem):
            return pltpu.async_copy(
                emb.at[e2.at[pl.ds(c * CK, CK)]], buf, sem)

        g_e(0, EA, sF)

        def pair(i, _):
            c0 = 2 * i
            c1 = 2 * i + 1
            heb = g_e(c1, EB, sFB_)
            pltpu.make_async_copy(
                emb.at[e2.at[pl.ds(0, CK)]], EA, sF).wait()
            _seg_reduce(EA, estage, c0 * (CK // S), DE // 16)

            @pl.when(i < NP - 1)
            def _():
                g_e(c0 + 2, EA, sF)

            heb.wait()
            _seg_reduce(EB, estage, c1 * (CK // S), DE // 16)
            return 0

        lax.fori_loop(0, NP, pair, 0)
        pltpu.sync_copy(estage, se2_o.at[pl.ds(wid * P1, P1)])

    return functools.partial(
        pl.kernel, mesh=mesh, out_type=out_type, scratch_types=scratch,
        compiler_params=_SC_PARAMS)(body)


def _repack_body(x_ref, o_ref):
    Rw, CB = x_ref.shape
    t3 = x_ref[...].T.reshape(CB // 8, 8, Rw)
    for s in range(8):
        o_ref[:, s * Rw:(s + 1) * Rw] = t3[:, s, :]


def _prep_transpose(xT, blk_c):
    Rw, C = xT.shape
    nsteps = C // blk_c
    return pl.pallas_call(
        _repack_body,
        grid=(nsteps,),
        in_specs=[pl.BlockSpec((Rw, blk_c), lambda i: (0, i))],
        out_specs=pl.BlockSpec((blk_c // 8, Rw * 8), lambda i: (i, 0)),
        out_shape=jax.ShapeDtypeStruct((C // 8, Rw * 8), xT.dtype),
    )(xT)


def _prep_transpose4(xTs):
    Rw, C = xTs[0].shape

    def body(x0, x1, x2, x3, o0, o1, o2, o3):
        for x_ref, o_ref in ((x0, o0), (x1, o1), (x2, o2), (x3, o3)):
            _repack_body(x_ref, o_ref)

    outs = pl.pallas_call(
        body,
        out_shape=[jax.ShapeDtypeStruct((C // 8, Rw * 8), x.dtype)
                   for x in xTs],
    )(*xTs)
    return outs


BB = 128
GRID = B // BB


def _tc_body(g0, g1_0, s2_0, e1_0, se2_0, g1_1, s2_1, e1_1, se2_1,
             Wp, bp,
             Ws00, Wn00, ba00, We00, be00, Ws01, Wn01, ba01,
             Ws10, Wn10, ba10, We10, be10, Ws11, Wn11, ba11,
             Wfc, bfc, out):
    inv_s = 1.0 / S

    def gmean(x):
        return jnp.sum(x.reshape(BB, S, x.shape[-1]), axis=1) * inv_s

    def grepeat(x):
        return jnp.broadcast_to(x[:, None, :], (BB, S, x.shape[-1])).reshape(BB * S, x.shape[-1])

    wp = Wp[...]
    bpv = bp[...]
    p0 = jnp.dot(g0[...], wp, preferred_element_type=F32) + bpv

    acc = None
    for g1, s2, e1, se2, Ws0, Wn0, ba0, We0, be0, Ws1, Wn1, ba1 in (
        (g1_0, s2_0, e1_0, se2_0, Ws00, Wn00, ba00, We00, be00, Ws01, Wn01, ba01),
        (g1_1, s2_1, e1_1, se2_1, Ws10, Wn10, ba10, We10, be10, Ws11, Wn11, ba11),
    ):
        wn0 = Wn0[...]
        wn1 = Wn1[...]
        we0 = We0[...]
        p1 = jnp.dot(g1[...], wp, preferred_element_type=F32) + bpv
        m2 = jnp.dot(s2[...] * inv_s, wp, preferred_element_type=F32) + bpv
        se2m = se2[...] * inv_s
        e1v = e1[...]
        f1 = jnp.maximum(
            jnp.dot(p1, Ws0[...], preferred_element_type=F32)
            + jnp.dot(m2, wn0[:D], preferred_element_type=F32)
            + jnp.dot(se2m, wn0[D:], preferred_element_type=F32) + ba0[...], 0.0)
        f0 = jnp.maximum(
            jnp.dot(p0, Ws0[...], preferred_element_type=F32)
            + jnp.dot(gmean(p1), wn0[:D], preferred_element_type=F32)
            + jnp.dot(gmean(e1v), wn0[D:], preferred_element_type=F32) + ba0[...], 0.0)
        e0n = jnp.maximum(
            jnp.dot(grepeat(f0), we0[:D], preferred_element_type=F32)
            + jnp.dot(f1, we0[D:2 * D], preferred_element_type=F32)
            + jnp.dot(e1v, we0[2 * D:], preferred_element_type=F32) + be0[...], 0.0)
        o = jnp.maximum(
            jnp.dot(f0, Ws1[...], preferred_element_type=F32)
            + jnp.dot(gmean(f1), wn1[:D], preferred_element_type=F32)
            + jnp.dot(gmean(e0n), wn1[D:], preferred_element_type=F32) + ba1[...], 0.0)
        acc = o if acc is None else acc + o

    res = acc * 0.5
    nrm = jnp.sqrt(jnp.sum(res * res, axis=1, keepdims=True))
    res = res / jnp.maximum(nrm, 1e-12)
    out[...] = jnp.dot(res, Wfc[...], preferred_element_type=F32) + bfc[...]


def _tc_call(g0, g1_0, s2_0, e1_0, se2_0, g1_1, s2_1, e1_1, se2_1, *ws):
    bspec_b = pl.BlockSpec((BB, D), lambda i: (i, 0))
    bspec_n = pl.BlockSpec((BB * S, D), lambda i: (i, 0))
    bspec_be = pl.BlockSpec((BB * S, DE), lambda i: (i, 0))
    full = lambda a: pl.BlockSpec(a.shape, lambda i: tuple(0 for _ in a.shape))
    in_specs = [bspec_b, bspec_n, bspec_n, bspec_be, bspec_be,
                bspec_n, bspec_n, bspec_be, bspec_be] + [full(w) for w in ws]
    return pl.pallas_call(
        _tc_body,
        grid=(GRID,),
        in_specs=in_specs,
        out_specs=pl.BlockSpec((BB, 8), lambda i: (i, 0)),
        out_shape=jax.ShapeDtypeStruct((B, 8), F32),
    )(g0, g1_0, s2_0, e1_0, se2_0, g1_1, s2_1, e1_1, se2_1, *ws)


def kernel(ids, feats, adj_0, adj_edge_0, edge_emb_0, adj_1, adj_edge_1,
           edge_emb_1, W_prep, b_prep,
           W_self_0_0, W_neigh_0_0, b_agg_0_0, W_edge_0_0, b_edge_0_0,
           W_self_0_1, W_neigh_0_1, b_agg_0_1, W_edge_0_1, b_edge_0_1,
           W_self_1_0, W_neigh_1_0, b_agg_1_0, W_edge_1_0, b_edge_1_0,
           W_self_1_1, W_neigh_1_1, b_agg_1_1, W_edge_1_1, b_edge_1_1,
           W_fc, b_fc):
    def repack(t, blk_c):
        n, w = t.shape
        return _prep_transpose(jnp.swapaxes(t, 0, 1), blk_c).reshape(n, w)

    adjs = _prep_transpose4([jnp.swapaxes(t, 0, 1) for t in
                             (adj_0, adj_edge_0, adj_1, adj_edge_1)])
    adj_0c, adje_0c, adj_1c, adje_1c = (
        t.reshape(N_NODES_, 32) for t in adjs)
    emb_0c = repack(edge_emb_0, 12800)
    emb_1c = repack(edge_emb_1, 12800)

    sc_a = _sc_feats_fn()
    (g0, g1_0, s2_0, g1_1, s2_1, e1i_0, e2i_0, e1i_1, e2i_1) = sc_a(
        ids, feats, adj_0c, adje_0c, adj_1c, adje_1c)
    sc_b = _sc_edges_fn()
    (e1_0, se2_0) = sc_b(emb_0c, e1i_0, e2i_0)
    (e1_1, se2_1) = sc_b(emb_1c, e1i_1, e2i_1)
    return _tc_call(
        g0, g1_0, s2_0, e1_0, se2_0, g1_1, s2_1, e1_1, se2_1,
        W_prep, b_prep,
        W_self_0_0, W_neigh_0_0, b_agg_0_0, W_edge_0_0, b_edge_0_0,
        W_self_0_1, W_neigh_0_1, b_agg_0_1,
        W_self_1_0, W_neigh_1_0, b_agg_1_0, W_edge_1_0, b_edge_1_0,
        W_self_1_1, W_neigh_1_1, b_agg_1_1,
        W_fc, b_fc)
